# baseline (speedup 1.0000x reference)
import jax
import jax.numpy as jnp
from jax import lax
from jax.experimental import pallas as pl
from jax.experimental.pallas import tpu as pltpu
from jax.experimental.pallas import tpu_sc as plsc

B = 4
CIN = 384
H = 200
W = 176
HW = H * W
NUM_CLASS = 4
A = 6
N_POS = 512
N_NEG = 4096
M = N_POS + N_NEG
ROWS = B * HW

NC, NS = 2, 16
NW = NC * NS
SPW = M // NW


def _sc_rows(table, idx):
  mesh = plsc.VectorSubcoreMesh(
      core_axis_name="c", subcore_axis_name="s", num_cores=NC, num_subcores=NS)

  def body(tab_h, idx_h, out_h, idx_v, g_v, sem):
    wid = lax.axis_index("s") * NC + lax.axis_index("c")
    s0 = wid * SPW
    pltpu.sync_copy(idx_h.at[pl.ds(s0, SPW)], idx_v)
    cps = [
        pltpu.async_copy(tab_h.at[idx_v.at[pl.ds(0, 128)]],
                         g_v.at[pl.ds(0, 128)], sem),
        pltpu.async_copy(tab_h.at[idx_v.at[pl.ds(128, SPW - 128)]],
                         g_v.at[pl.ds(128, SPW - 128)], sem),
    ]
    for cp in cps:
      cp.wait()
    pltpu.sync_copy(g_v, out_h.at[pl.ds(s0, SPW)])

  f = pl.kernel(
      body,
      out_type=jax.ShapeDtypeStruct((M, 256), jnp.int32),
      mesh=mesh,
      scratch_types=[
          pltpu.VMEM((SPW,), jnp.int32),
          pltpu.VMEM((SPW, 256), jnp.int32),
          pltpu.SemaphoreType.DMA,
      ],
  )
  return f(table, idx)


def _loss_body(x_ref, w_ref, b_ref, a_ref, lbl_ref, rl_ref, out_ref):
  x = x_ref[...]
  logits = jnp.dot(x, w_ref[...],
                   preferred_element_type=jnp.float32) + b_ref[...]
  abase = a_ref[...] * 16
  lane = lax.broadcasted_iota(jnp.int32, (M, 128), 1)
  cls = []
  for c in range(NUM_CLASS):
    sel = lane == (abase + c)
    cls.append(jnp.sum(jnp.where(sel, logits, 0.0), axis=1, keepdims=True))
  mx = jnp.maximum(jnp.maximum(cls[0], cls[1]), jnp.maximum(cls[2], cls[3]))
  se = (jnp.exp(cls[0] - mx) + jnp.exp(cls[1] - mx)
        + jnp.exp(cls[2] - mx) + jnp.exp(cls[3] - mx))
  lse = jnp.log(se) + mx
  lbl = lbl_ref[...]
  picked = sum(jnp.where(lbl == c, cls[c], 0.0) for c in range(NUM_CLASS))
  cls_loss = jnp.mean(lse - picked)

  logits_p = logits[N_NEG:, :]
  ab_p = a_ref[...][N_NEG:, :] * 16
  lane_p = lax.broadcasted_iota(jnp.int32, (N_POS, 128), 1)
  racc = jnp.zeros((), jnp.float32)
  for j in range(7):
    sel = lane_p == (ab_p + 4 + j)
    pj = jnp.sum(jnp.where(sel, logits_p, 0.0), axis=1, keepdims=True)
    d = pj - rl_ref[...][:, j:j + 1]
    ad = jnp.abs(d)
    racc = racc + jnp.sum(jnp.where(ad < 1.0, 0.5 * d * d, ad - 0.5))
  reg_loss = racc / (N_POS * 7)
  out_ref[...] = jnp.full((1, 1), cls_loss + 2.0 * reg_loss, jnp.float32)


def kernel(inputs, pos_batch_ids, pos_bbox_ids, neg_batch_ids, neg_bbox_ids,
           cls_labels, reg_labels, Wc, bc, Wr, br):
  a_l = jnp.arange(128, dtype=jnp.int32) // 16
  k_l = jnp.arange(128, dtype=jnp.int32) % 16
  valid = (a_l < A) & (k_l < NUM_CLASS + 7)
  src = jnp.where(k_l < NUM_CLASS, k_l * A + a_l,
                  NUM_CLASS * A + (k_l - NUM_CLASS) * A + a_l)
  src = jnp.where(valid, src, 0)
  wcat = jnp.concatenate([Wc, Wr], axis=0)
  wp = jnp.where(valid[None, :], wcat.T[:, src], 0.0).astype(jnp.bfloat16)
  bp = jnp.where(valid, jnp.concatenate([bc, br])[src], 0.0).reshape(1, 128)

  xb = jnp.transpose(inputs, (0, 2, 3, 1)).astype(jnp.bfloat16).reshape(ROWS, CIN)
  xp = jnp.concatenate([xb, jnp.zeros((ROWS, 128), jnp.bfloat16)], axis=1)
  xt32 = lax.bitcast_convert_type(xp.reshape(ROWS, 256, 2), jnp.int32)

  all_b = jnp.concatenate([neg_batch_ids, pos_batch_ids]).astype(jnp.int32)
  all_t = jnp.concatenate([neg_bbox_ids, pos_bbox_ids]).astype(jnp.int32)
  a_sel = all_t // HW
  row_idx = all_b * HW + all_t % HW

  g32 = _sc_rows(xt32, row_idx)
  x2 = lax.bitcast_convert_type(
      g32, jnp.bfloat16).reshape(M, 512)[:, :CIN]

  rl_pad = jnp.concatenate(
      [reg_labels, jnp.zeros((N_POS, 1), jnp.float32)], axis=1)
  res = pl.pallas_call(
      _loss_body,
      out_shape=jax.ShapeDtypeStruct((1, 1), jnp.float32),
  )(x2, wp, bp, a_sel.reshape(M, 1), cls_labels.astype(jnp.int32).reshape(M, 1),
    rl_pad)
  return res[0, 0]


# speedup vs baseline: 30.2929x; 30.2929x over previous
import jax
import jax.numpy as jnp
from jax import lax
from jax.experimental import pallas as pl
from jax.experimental.pallas import tpu as pltpu
from jax.experimental.pallas import tpu_sc as plsc

B = 4
CIN = 384
H = 200
W = 176
HW = H * W
NUM_CLASS = 4
A = 6
N_POS = 512
N_NEG = 4096
M = N_POS + N_NEG
ROWS = B * HW

NC, NS = 2, 16
NW = NC * NS
SPW = M // NW


def _sc_rows(table, idx):
  mesh = plsc.VectorSubcoreMesh(
      core_axis_name="c", subcore_axis_name="s", num_cores=NC, num_subcores=NS)

  def body(tab_h, idx_h, out_h, idx_v, g_v, sem):
    wid = lax.axis_index("s") * NC + lax.axis_index("c")
    s0 = wid * SPW
    pltpu.sync_copy(idx_h.at[pl.ds(s0, SPW)], idx_v)
    cps = [
        pltpu.async_copy(tab_h.at[idx_v.at[pl.ds(0, 128)]],
                         g_v.at[pl.ds(0, 128)], sem),
        pltpu.async_copy(tab_h.at[idx_v.at[pl.ds(128, SPW - 128)]],
                         g_v.at[pl.ds(128, SPW - 128)], sem),
    ]
    for cp in cps:
      cp.wait()
    pltpu.sync_copy(g_v, out_h.at[pl.ds(s0, SPW)])

  f = pl.kernel(
      body,
      out_type=jax.ShapeDtypeStruct((M, CIN), jnp.float32),
      mesh=mesh,
      scratch_types=[
          pltpu.VMEM((SPW,), jnp.int32),
          pltpu.VMEM((SPW, CIN), jnp.float32),
          pltpu.SemaphoreType.DMA,
      ],
  )
  return f(table, idx)


def _loss_body(x_ref, w_ref, b_ref, a_ref, lbl_ref, rl_ref, out_ref):
  x = x_ref[...].astype(jnp.bfloat16)
  logits = jnp.dot(x, w_ref[...],
                   preferred_element_type=jnp.float32) + b_ref[...]
  abase = a_ref[...] * 16
  lane = lax.broadcasted_iota(jnp.int32, (M, 128), 1)
  cls = []
  for c in range(NUM_CLASS):
    sel = lane == (abase + c)
    cls.append(jnp.sum(jnp.where(sel, logits, 0.0), axis=1, keepdims=True))
  mx = jnp.maximum(jnp.maximum(cls[0], cls[1]), jnp.maximum(cls[2], cls[3]))
  se = (jnp.exp(cls[0] - mx) + jnp.exp(cls[1] - mx)
        + jnp.exp(cls[2] - mx) + jnp.exp(cls[3] - mx))
  lse = jnp.log(se) + mx
  lbl = lbl_ref[...]
  picked = sum(jnp.where(lbl == c, cls[c], 0.0) for c in range(NUM_CLASS))
  cls_loss = jnp.mean(lse - picked)

  logits_p = logits[N_NEG:, :]
  ab_p = a_ref[...][N_NEG:, :] * 16
  lane_p = lax.broadcasted_iota(jnp.int32, (N_POS, 128), 1)
  racc = jnp.zeros((), jnp.float32)
  for j in range(7):
    sel = lane_p == (ab_p + 4 + j)
    pj = jnp.sum(jnp.where(sel, logits_p, 0.0), axis=1, keepdims=True)
    d = pj - rl_ref[...][:, j:j + 1]
    ad = jnp.abs(d)
    racc = racc + jnp.sum(jnp.where(ad < 1.0, 0.5 * d * d, ad - 0.5))
  reg_loss = racc / (N_POS * 7)
  out_ref[...] = jnp.full((1, 1), cls_loss + 2.0 * reg_loss, jnp.float32)


def kernel(inputs, pos_batch_ids, pos_bbox_ids, neg_batch_ids, neg_bbox_ids,
           cls_labels, reg_labels, Wc, bc, Wr, br):
  a_l = jnp.arange(128, dtype=jnp.int32) // 16
  k_l = jnp.arange(128, dtype=jnp.int32) % 16
  valid = (a_l < A) & (k_l < NUM_CLASS + 7)
  src = jnp.where(k_l < NUM_CLASS, k_l * A + a_l,
                  NUM_CLASS * A + (k_l - NUM_CLASS) * A + a_l)
  src = jnp.where(valid, src, 0)
  wcat = jnp.concatenate([Wc, Wr], axis=0)
  wp = jnp.where(valid[None, :], wcat.T[:, src], 0.0).astype(jnp.bfloat16)
  bp = jnp.where(valid, jnp.concatenate([bc, br])[src], 0.0).reshape(1, 128)

  xt = jnp.transpose(inputs, (0, 2, 3, 1)).reshape(ROWS, CIN)

  all_b = jnp.concatenate([neg_batch_ids, pos_batch_ids]).astype(jnp.int32)
  all_t = jnp.concatenate([neg_bbox_ids, pos_bbox_ids]).astype(jnp.int32)
  a_sel = all_t // HW
  row_idx = all_b * HW + all_t % HW

  x2 = _sc_rows(xt, row_idx)

  rl_pad = jnp.concatenate(
      [reg_labels, jnp.zeros((N_POS, 1), jnp.float32)], axis=1)
  res = pl.pallas_call(
      _loss_body,
      out_shape=jax.ShapeDtypeStruct((1, 1), jnp.float32),
  )(x2, wp, bp, a_sel.reshape(M, 1), cls_labels.astype(jnp.int32).reshape(M, 1),
    rl_pad)
  return res[0, 0]
